# baseline (speedup 1.0000x reference)
import functools

import jax
import jax.numpy as jnp
from jax import lax
from jax.experimental import pallas as pl
from jax.experimental.pallas import tpu as pltpu
from jax.experimental.pallas import tpu_sc as plsc

_VOCAB = 1000000
_D = 16
_B, _S = 4096, 200
_NTOK = _B * _S
_BLK = 128
_NROWS = _NTOK // _BLK
_P2 = 257 * 257
_P3 = (257 ** 3) % _VOCAB

_info = plsc.get_sparse_core_info()
_NC, _NS = _info.num_cores, _info.num_subcores
_NW = _NC * _NS
_NB = _NROWS // _NW

_MAIN_COLS = 7808
_COLS_PER_W = _MAIN_COLS // _NW
_K = 4
_SC_PER_W = _COLS_PER_W // _K
_TAIL_ROW0 = _MAIN_COLS * 128
_PACK_ROWS = 125000


def _hash_body(x0, x1, x2, x3, h2o, h3o, h4o):
    h2 = x1[...] * 257 + x0[...]
    h3 = (h2 + x2[...] * _P2) % _VOCAB
    h4 = (h3 + x3[...] * _P3) % _VOCAB
    h2o[...] = h2
    h3o[...] = h3
    h4o[...] = h4


def _hash_tc(x0, x1, x2, x3):
    grid = 8
    rows = _B // grid
    spec = pl.BlockSpec((rows, _S), lambda i: (i, jnp.int32(0)))
    return pl.pallas_call(
        _hash_body,
        grid=(grid,),
        in_specs=[spec] * 4,
        out_specs=[spec] * 3,
        out_shape=[jax.ShapeDtypeStruct((_B, _S), jnp.int32)] * 3,
    )(x0, x1, x2, x3)


@functools.partial(
    pl.kernel,
    out_type=[jax.ShapeDtypeStruct((_PACK_ROWS, 128), jnp.float32)] * 3,
    mesh=plsc.VectorSubcoreMesh(core_axis_name="c", subcore_axis_name="s"),
    compiler_params=pltpu.CompilerParams(
        use_tc_tiling_on_sc=True, needs_layout_passes=False),
    scratch_types=[
        pltpu.VMEM((_K, 16, 128), jnp.float32),
        pltpu.VMEM((_K * 16, 128), jnp.float32),
        pltpu.SemaphoreType.DMA,
    ],
)
def _sc_format(t2t, t3t, t4t, tl2, tl3, tl4, f2, f3, f4, xv, ov, si):
    i32 = jnp.int32
    wid = lax.axis_index("s") * i32(_NC) + lax.axis_index("c")
    iota = lax.iota(jnp.int32, 16)

    def transform_col(k_idx, c2, row):
        base_c = c2 * i32(8)
        for u in range(8):
            vec = plsc.load_gather(xv, [k_idx, iota, jnp.full((16,), base_c, jnp.int32) + i32(u)])
            ov[row, pl.ds(u * 16, 16)] = vec

    def do_table(src_h, fmt_h):
        def sc_body(g, carry):
            col0 = (wid * i32(_SC_PER_W) + g) * i32(_K)
            for k in range(_K):
                pltpu.async_copy(
                    src_h.at[:, pl.ds((col0 + i32(k)) * 128, 128)], xv.at[i32(k)], si)
            for k in range(_K):
                pltpu.make_async_copy(
                    src_h.at[:, pl.ds((col0 + i32(k)) * 128, 128)], xv.at[i32(k)], si).wait()

            def c2_body(c2, carry2):
                for k in range(_K):
                    transform_col(jnp.full((16,), k, jnp.int32), c2,
                                  i32(k * 16) + c2)
                return carry2

            lax.fori_loop(i32(0), i32(16), c2_body, i32(0))
            pltpu.sync_copy(ov, fmt_h.at[pl.ds(col0 * 16, _K * 16)])
            return carry

        lax.fori_loop(i32(0), i32(_SC_PER_W), sc_body, i32(0))

    def do_tail(tail_h, fmt_h):
        @pl.when(wid < i32(5))
        def _():
            pltpu.sync_copy(tail_h.at[:, pl.ds(wid * i32(128), 128)], xv.at[i32(0)])

            def c2_body(c2, carry2):
                transform_col(jnp.full((16,), 0, jnp.int32), c2, c2)
                return carry2

            lax.fori_loop(i32(0), i32(16), c2_body, i32(0))
            pbase = i32(_TAIL_ROW0 // 8) + wid * i32(16)

            @pl.when(wid < i32(4))
            def _():
                pltpu.sync_copy(ov.at[pl.ds(0, 16)], fmt_h.at[pl.ds(pbase, 16)])

            @pl.when(wid == i32(4))
            def _():
                pltpu.sync_copy(ov.at[pl.ds(0, 8)], fmt_h.at[pl.ds(pbase, 8)])

    do_table(t2t, f2)
    do_table(t3t, f3)
    do_table(t4t, f4)
    do_tail(tl2, f2)
    do_tail(tl3, f3)
    do_tail(tl4, f4)


@functools.partial(
    pl.kernel,
    out_type=jax.ShapeDtypeStruct((_NROWS, _BLK, _D), jnp.float32),
    mesh=plsc.VectorSubcoreMesh(core_axis_name="c", subcore_axis_name="s"),
    compiler_params=pltpu.CompilerParams(use_tc_tiling_on_sc=False),
    scratch_types=[
        pltpu.VMEM((_NB, _BLK), jnp.int32),
        pltpu.VMEM((_NB, _BLK), jnp.int32),
        pltpu.VMEM((_NB, _BLK), jnp.int32),
        pltpu.VMEM((_BLK, _D), jnp.float32),
        pltpu.VMEM((_BLK, _D), jnp.float32),
        pltpu.VMEM((_BLK, _D), jnp.float32),
        pltpu.VMEM((_BLK, _D), jnp.float32),
        pltpu.VMEM((_BLK, _D), jnp.float32),
        pltpu.VMEM((_BLK, _D), jnp.float32),
        pltpu.SemaphoreType.DMA,
        pltpu.SemaphoreType.DMA,
        pltpu.SemaphoreType.DMA,
        pltpu.SemaphoreType.DMA,
    ],
)
def _sc_embed(h2_h, h3_h, h4_h, t2_h, t3_h, t4_h, out_h,
              vi2, vi3, vi4,
              ra2, ra3, ra4, rb2, rb3, rb4,
              sga, sgb, soa, sob):
    i32 = jnp.int32
    wid = lax.axis_index("s") * i32(_NC) + lax.axis_index("c")
    row0 = wid * i32(_NB)

    pltpu.sync_copy(h2_h.at[pl.ds(row0, _NB)], vi2)
    pltpu.sync_copy(h3_h.at[pl.ds(row0, _NB)], vi3)
    pltpu.sync_copy(h4_h.at[pl.ds(row0, _NB)], vi4)

    def fire_gathers(b, r2, r3, r4, sg):
        pltpu.async_copy(t2_h.at[vi2.at[b]], r2, sg)
        pltpu.async_copy(t3_h.at[vi3.at[b]], r3, sg)
        pltpu.async_copy(t4_h.at[vi4.at[b]], r4, sg)

    def wait_gathers(b, r2, r3, r4, sg):
        pltpu.make_async_copy(t2_h.at[vi2.at[b]], r2, sg).wait()
        pltpu.make_async_copy(t3_h.at[vi3.at[b]], r3, sg).wait()
        pltpu.make_async_copy(t4_h.at[vi4.at[b]], r4, sg).wait()

    def step(b, r2, r3, r4, sg, q2, q3, q4, sq, so_other, so_mine):
        @pl.when(b >= i32(1))
        def _():
            pltpu.make_async_copy(q2, out_h.at[row0 + b - 1], so_other).wait()

        @pl.when(b + 1 < i32(_NB))
        def _():
            fire_gathers(b + 1, q2, q3, q4, sq)

        wait_gathers(b, r2, r3, r4, sg)

        def acc_body(i, carry):
            base = i * i32(8)
            for k in range(8):
                r2[base + k, :] = r2[base + k, :] + r3[base + k, :] + r4[base + k, :]
            return carry

        lax.fori_loop(i32(0), i32(_BLK // 8), acc_body, i32(0))
        pltpu.async_copy(r2, out_h.at[row0 + b], so_mine)

    fire_gathers(i32(0), ra2, ra3, ra4, sga)

    def outer(b2, carry):
        b = b2 * i32(2)
        step(b, ra2, ra3, ra4, sga, rb2, rb3, rb4, sgb, sob, soa)
        step(b + i32(1), rb2, rb3, rb4, sgb, ra2, ra3, ra4, sga, soa, sob)
        return carry

    lax.fori_loop(i32(0), i32(_NB // 2), outer, i32(0))
    pltpu.make_async_copy(rb2, out_h.at[row0 + i32(_NB - 1)], sob).wait()


def kernel(inputs, table_2, table_3, table_4):
    x = inputs.astype(jnp.int32)
    x1 = jnp.pad(x, ((0, 0), (1, 0)))[:, :_S]
    x2 = jnp.pad(x, ((0, 0), (2, 0)))[:, :_S]
    x3 = jnp.pad(x, ((0, 0), (3, 0)))[:, :_S]
    h2, h3, h4 = _hash_tc(x, x1, x2, x3)
    tails = [
        jnp.pad(t[_TAIL_ROW0:].T, ((0, 0), (0, 64)))
        for t in (table_2, table_3, table_4)
    ]
    f2, f3, f4 = _sc_format(
        table_2.T, table_3.T, table_4.T, tails[0], tails[1], tails[2])
    out = _sc_embed(
        h2.reshape(_NROWS, _BLK),
        h3.reshape(_NROWS, _BLK),
        h4.reshape(_NROWS, _BLK),
        f2.reshape(_VOCAB, _D),
        f3.reshape(_VOCAB, _D),
        f4.reshape(_VOCAB, _D),
    )
    return out.reshape(_B, _S, _D)


# speedup vs baseline: 1.8343x; 1.8343x over previous
import functools

import jax
import jax.numpy as jnp
from jax import lax
from jax.experimental import pallas as pl
from jax.experimental.pallas import tpu as pltpu
from jax.experimental.pallas import tpu_sc as plsc

_VOCAB = 1000000
_D = 16
_B, _S = 4096, 200
_NTOK = _B * _S
_BLK = 128
_NROWS = _NTOK // _BLK
_P2 = 257 * 257
_P3 = (257 ** 3) % _VOCAB

_info = plsc.get_sparse_core_info()
_NC, _NS = _info.num_cores, _info.num_subcores
_NW = _NC * _NS
_NB = _NROWS // _NW

_MAIN_COLS = 7808
_COLS_PER_W = _MAIN_COLS // _NW
_K = 4
_SC_PER_W = _COLS_PER_W // _K
_TAIL_ROW0 = _MAIN_COLS * 128
_PACK_ROWS = 125000


def _hash_body(x0, x1, x2, x3, h2o, h3o, h4o):
    h2 = x1[...] * 257 + x0[...]
    h3 = (h2 + x2[...] * _P2) % _VOCAB
    h4 = (h3 + x3[...] * _P3) % _VOCAB
    h2o[...] = h2
    h3o[...] = h3
    h4o[...] = h4


def _hash_tc(x0, x1, x2, x3):
    grid = 8
    rows = _B // grid
    spec = pl.BlockSpec((rows, _S), lambda i: (i, jnp.int32(0)))
    return pl.pallas_call(
        _hash_body,
        grid=(grid,),
        in_specs=[spec] * 4,
        out_specs=[spec] * 3,
        out_shape=[jax.ShapeDtypeStruct((_B, _S), jnp.int32)] * 3,
    )(x0, x1, x2, x3)


_CHUNK_WORDS = _K * 16 * 128


@functools.partial(
    pl.kernel,
    out_type=[jax.ShapeDtypeStruct((_VOCAB * _D,), jnp.float32)] * 3,
    mesh=plsc.VectorSubcoreMesh(core_axis_name="c", subcore_axis_name="s"),
    compiler_params=pltpu.CompilerParams(
        use_tc_tiling_on_sc=True, needs_layout_passes=False),
    scratch_types=[
        pltpu.VMEM((_K, 16, 128), jnp.float32),
        pltpu.VMEM((_K, 16, 128), jnp.float32),
        pltpu.VMEM((_CHUNK_WORDS,), jnp.float32),
        pltpu.VMEM((_CHUNK_WORDS,), jnp.float32),
        pltpu.SemaphoreType.DMA,
        pltpu.SemaphoreType.DMA,
        pltpu.SemaphoreType.DMA,
        pltpu.SemaphoreType.DMA,
    ],
)
def _sc_format(t2t, t3t, t4t, tl2, tl3, tl4, f2, f3, f4,
               xva, xvb, ova, ovb, sia, sib, soa, sob):
    i32 = jnp.int32
    wid = lax.axis_index("s") * i32(_NC) + lax.axis_index("c")
    iota = lax.iota(jnp.int32, 16)
    v0 = (iota // i32(8)) * i32(128) + (iota % i32(8)) * i32(16)

    def fire_loads(src_h, g, xv, si):
        col0 = (wid * i32(_SC_PER_W) + g) * i32(_K)
        for k in range(_K):
            pltpu.async_copy(
                src_h.at[:, pl.ds((col0 + i32(k)) * 128, 128)], xv.at[i32(k)], si)

    def wait_loads(src_h, g, xv, si):
        col0 = (wid * i32(_SC_PER_W) + g) * i32(_K)
        for k in range(_K):
            pltpu.make_async_copy(
                src_h.at[:, pl.ds((col0 + i32(k)) * 128, 128)], xv.at[i32(k)], si).wait()

    def transform(xv, ov):
        def d_body(d, carry):
            for k in range(_K):
                for c0 in range(8):
                    vec = xv[i32(k), d, pl.ds(c0 * 16, 16)]
                    idx = v0 + (d + i32(k * 2048 + c0 * 256))
                    plsc.store_scatter(ov, [idx], vec)
            return carry

        lax.fori_loop(i32(0), i32(16), d_body, i32(0))

    def wait_out(fmt_h, ov, so):
        pltpu.make_async_copy(ov, fmt_h.at[pl.ds(i32(0), _CHUNK_WORDS)], so).wait()

    def do_table(src_h, fmt_h):
        def step(g, xv, ov, si, so):
            wait_loads(src_h, g, xv, si)

            @pl.when(g >= i32(2))
            def _():
                wait_out(fmt_h, ov, so)

            transform(xv, ov)

            @pl.when(g + i32(2) < i32(_SC_PER_W))
            def _():
                fire_loads(src_h, g + i32(2), xv, si)

            base = (wid * i32(_SC_PER_W) + g) * i32(_CHUNK_WORDS)
            pltpu.async_copy(ov, fmt_h.at[pl.ds(base, _CHUNK_WORDS)], so)

        fire_loads(src_h, i32(0), xva, sia)
        fire_loads(src_h, i32(1), xvb, sib)

        def pair(g2, carry):
            g = g2 * i32(2)
            step(g, xva, ova, sia, soa)
            step(g + i32(1), xvb, ovb, sib, sob)
            return carry

        lax.fori_loop(i32(0), i32(_SC_PER_W // 2), pair, i32(0))
        step(i32(_SC_PER_W - 1), xva, ova, sia, soa)
        wait_out(fmt_h, ova, soa)
        wait_out(fmt_h, ovb, sob)

    def do_tail(tail_h, fmt_h):
        @pl.when(wid < i32(5))
        def _():
            pltpu.sync_copy(tail_h.at[:, pl.ds(wid * i32(128), 128)], xva.at[i32(0)])

            def d_body(d, carry):
                for c0 in range(8):
                    vec = xva[i32(0), d, pl.ds(c0 * 16, 16)]
                    idx = v0 + (d + i32(c0 * 256))
                    plsc.store_scatter(ova, [idx], vec)
                return carry

            lax.fori_loop(i32(0), i32(16), d_body, i32(0))
            wbase = i32(_TAIL_ROW0 * _D) + wid * i32(2048)

            @pl.when(wid < i32(4))
            def _():
                pltpu.sync_copy(ova.at[pl.ds(i32(0), 2048)],
                                fmt_h.at[pl.ds(wbase, 2048)])

            @pl.when(wid == i32(4))
            def _():
                pltpu.sync_copy(ova.at[pl.ds(i32(0), 1024)],
                                fmt_h.at[pl.ds(wbase, 1024)])

    do_table(t2t, f2)
    do_table(t3t, f3)
    do_table(t4t, f4)
    do_tail(tl2, f2)
    do_tail(tl3, f3)
    do_tail(tl4, f4)


@functools.partial(
    pl.kernel,
    out_type=jax.ShapeDtypeStruct((_NROWS, _BLK, _D), jnp.float32),
    mesh=plsc.VectorSubcoreMesh(core_axis_name="c", subcore_axis_name="s"),
    compiler_params=pltpu.CompilerParams(use_tc_tiling_on_sc=False),
    scratch_types=[
        pltpu.VMEM((_NB, _BLK), jnp.int32),
        pltpu.VMEM((_NB, _BLK), jnp.int32),
        pltpu.VMEM((_NB, _BLK), jnp.int32),
        pltpu.VMEM((_BLK, _D), jnp.float32),
        pltpu.VMEM((_BLK, _D), jnp.float32),
        pltpu.VMEM((_BLK, _D), jnp.float32),
        pltpu.VMEM((_BLK, _D), jnp.float32),
        pltpu.VMEM((_BLK, _D), jnp.float32),
        pltpu.VMEM((_BLK, _D), jnp.float32),
        pltpu.SemaphoreType.DMA,
        pltpu.SemaphoreType.DMA,
        pltpu.SemaphoreType.DMA,
        pltpu.SemaphoreType.DMA,
    ],
)
def _sc_embed(h2_h, h3_h, h4_h, t2_h, t3_h, t4_h, out_h,
              vi2, vi3, vi4,
              ra2, ra3, ra4, rb2, rb3, rb4,
              sga, sgb, soa, sob):
    i32 = jnp.int32
    wid = lax.axis_index("s") * i32(_NC) + lax.axis_index("c")
    row0 = wid * i32(_NB)

    pltpu.sync_copy(h2_h.at[pl.ds(row0, _NB)], vi2)
    pltpu.sync_copy(h3_h.at[pl.ds(row0, _NB)], vi3)
    pltpu.sync_copy(h4_h.at[pl.ds(row0, _NB)], vi4)

    def fire_gathers(b, r2, r3, r4, sg):
        pltpu.async_copy(t2_h.at[vi2.at[b]], r2, sg)
        pltpu.async_copy(t3_h.at[vi3.at[b]], r3, sg)
        pltpu.async_copy(t4_h.at[vi4.at[b]], r4, sg)

    def wait_gathers(b, r2, r3, r4, sg):
        pltpu.make_async_copy(t2_h.at[vi2.at[b]], r2, sg).wait()
        pltpu.make_async_copy(t3_h.at[vi3.at[b]], r3, sg).wait()
        pltpu.make_async_copy(t4_h.at[vi4.at[b]], r4, sg).wait()

    def step(b, r2, r3, r4, sg, q2, q3, q4, sq, so_other, so_mine):
        @pl.when(b >= i32(1))
        def _():
            pltpu.make_async_copy(q2, out_h.at[row0 + b - 1], so_other).wait()

        @pl.when(b + 1 < i32(_NB))
        def _():
            fire_gathers(b + 1, q2, q3, q4, sq)

        wait_gathers(b, r2, r3, r4, sg)

        def acc_body(i, carry):
            base = i * i32(8)
            for k in range(8):
                r2[base + k, :] = r2[base + k, :] + r3[base + k, :] + r4[base + k, :]
            return carry

        lax.fori_loop(i32(0), i32(_BLK // 8), acc_body, i32(0))
        pltpu.async_copy(r2, out_h.at[row0 + b], so_mine)

    fire_gathers(i32(0), ra2, ra3, ra4, sga)

    def outer(b2, carry):
        b = b2 * i32(2)
        step(b, ra2, ra3, ra4, sga, rb2, rb3, rb4, sgb, sob, soa)
        step(b + i32(1), rb2, rb3, rb4, sgb, ra2, ra3, ra4, sga, soa, sob)
        return carry

    lax.fori_loop(i32(0), i32(_NB // 2), outer, i32(0))
    pltpu.make_async_copy(rb2, out_h.at[row0 + i32(_NB - 1)], sob).wait()


def kernel(inputs, table_2, table_3, table_4):
    x = inputs.astype(jnp.int32)
    x1 = jnp.pad(x, ((0, 0), (1, 0)))[:, :_S]
    x2 = jnp.pad(x, ((0, 0), (2, 0)))[:, :_S]
    x3 = jnp.pad(x, ((0, 0), (3, 0)))[:, :_S]
    h2, h3, h4 = _hash_tc(x, x1, x2, x3)
    tails = [
        jnp.pad(t[_TAIL_ROW0:].T, ((0, 0), (0, 64)))
        for t in (table_2, table_3, table_4)
    ]
    f2, f3, f4 = _sc_format(
        table_2.T, table_3.T, table_4.T, tails[0], tails[1], tails[2])
    out = _sc_embed(
        h2.reshape(_NROWS, _BLK),
        h3.reshape(_NROWS, _BLK),
        h4.reshape(_NROWS, _BLK),
        f2.reshape(_VOCAB, _D),
        f3.reshape(_VOCAB, _D),
        f4.reshape(_VOCAB, _D),
    )
    return out.reshape(_B, _S, _D)


# speedup vs baseline: 1.8388x; 1.0025x over previous
import functools

import jax
import jax.numpy as jnp
from jax import lax
from jax.experimental import pallas as pl
from jax.experimental.pallas import tpu as pltpu
from jax.experimental.pallas import tpu_sc as plsc

_VOCAB = 1000000
_D = 16
_B, _S = 4096, 200
_NTOK = _B * _S
_BLK = 128
_NROWS = _NTOK // _BLK
_P2 = 257 * 257
_P3 = (257 ** 3) % _VOCAB

_info = plsc.get_sparse_core_info()
_NC, _NS = _info.num_cores, _info.num_subcores
_NW = _NC * _NS
_NB = _NROWS // _NW

_MAIN_COLS = 7808
_COLS_PER_W = _MAIN_COLS // _NW
_K = 4
_SC_PER_W = _COLS_PER_W // _K
_TAIL_ROW0 = _MAIN_COLS * 128
_PACK_ROWS = 125000


def _hash_body(x0, x1, x2, x3, h2o, h3o, h4o):
    h2 = x1[...] * 257 + x0[...]
    h3 = (h2 + x2[...] * _P2) % _VOCAB
    h4 = (h3 + x3[...] * _P3) % _VOCAB
    h2o[...] = h2
    h3o[...] = h3
    h4o[...] = h4


def _hash_tc(x0, x1, x2, x3):
    grid = 8
    rows = _B // grid
    spec = pl.BlockSpec((rows, _S), lambda i: (i, jnp.int32(0)))
    return pl.pallas_call(
        _hash_body,
        grid=(grid,),
        in_specs=[spec] * 4,
        out_specs=[spec] * 3,
        out_shape=[jax.ShapeDtypeStruct((_B, _S), jnp.int32)] * 3,
    )(x0, x1, x2, x3)


_CHUNK_WORDS = _K * 16 * 128


@functools.partial(
    pl.kernel,
    out_type=[jax.ShapeDtypeStruct((_VOCAB * _D,), jnp.float32)] * 3,
    mesh=plsc.VectorSubcoreMesh(core_axis_name="c", subcore_axis_name="s"),
    compiler_params=pltpu.CompilerParams(
        use_tc_tiling_on_sc=True, needs_layout_passes=False),
    scratch_types=[
        pltpu.VMEM((_K, 16, 128), jnp.float32),
        pltpu.VMEM((_K, 16, 128), jnp.float32),
        pltpu.VMEM((_CHUNK_WORDS,), jnp.float32),
        pltpu.VMEM((_CHUNK_WORDS,), jnp.float32),
        pltpu.SemaphoreType.DMA,
        pltpu.SemaphoreType.DMA,
        pltpu.SemaphoreType.DMA,
        pltpu.SemaphoreType.DMA,
    ],
)
def _sc_format(t2t, t3t, t4t, tl2, tl3, tl4, f2, f3, f4,
               xva, xvb, ova, ovb, sia, sib, soa, sob):
    i32 = jnp.int32
    wid = lax.axis_index("s") * i32(_NC) + lax.axis_index("c")
    iota = lax.iota(jnp.int32, 16)
    v0 = (iota // i32(8)) * i32(128) + (iota % i32(8)) * i32(16)

    def fire_loads(src_h, g, xv, si):
        col0 = (wid * i32(_SC_PER_W) + g) * i32(_K)
        for k in range(_K):
            pltpu.async_copy(
                src_h.at[:, pl.ds((col0 + i32(k)) * 128, 128)], xv.at[i32(k)], si)

    def wait_loads(src_h, g, xv, si):
        col0 = (wid * i32(_SC_PER_W) + g) * i32(_K)
        for k in range(_K):
            pltpu.make_async_copy(
                src_h.at[:, pl.ds((col0 + i32(k)) * 128, 128)], xv.at[i32(k)], si).wait()

    def transform(xv, ov):
        def d_body(d, carry):
            vd = v0 + d
            for k in range(_K):
                for c0 in range(8):
                    vec = xv[i32(k), d, pl.ds(c0 * 16, 16)]
                    idx = vd + i32(k * 2048 + c0 * 256)
                    plsc.store_scatter(ov, [idx], vec)
            return carry

        lax.fori_loop(i32(0), i32(16), d_body, i32(0))

    def wait_out(fmt_h, ov, so):
        pltpu.make_async_copy(ov, fmt_h.at[pl.ds(i32(0), _CHUNK_WORDS)], so).wait()

    def do_table(src_h, fmt_h):
        def step(g, xv, ov, si, so):
            wait_loads(src_h, g, xv, si)

            @pl.when(g >= i32(2))
            def _():
                wait_out(fmt_h, ov, so)

            transform(xv, ov)

            @pl.when(g + i32(2) < i32(_SC_PER_W))
            def _():
                fire_loads(src_h, g + i32(2), xv, si)

            base = (wid * i32(_SC_PER_W) + g) * i32(_CHUNK_WORDS)
            pltpu.async_copy(ov, fmt_h.at[pl.ds(base, _CHUNK_WORDS)], so)

        fire_loads(src_h, i32(0), xva, sia)
        fire_loads(src_h, i32(1), xvb, sib)

        def pair(g2, carry):
            g = g2 * i32(2)
            step(g, xva, ova, sia, soa)
            step(g + i32(1), xvb, ovb, sib, sob)
            return carry

        lax.fori_loop(i32(0), i32(_SC_PER_W // 2), pair, i32(0))
        step(i32(_SC_PER_W - 1), xva, ova, sia, soa)
        wait_out(fmt_h, ova, soa)
        wait_out(fmt_h, ovb, sob)

    def do_tail(tail_h, fmt_h):
        @pl.when(wid < i32(5))
        def _():
            pltpu.sync_copy(tail_h.at[:, pl.ds(wid * i32(128), 128)], xva.at[i32(0)])

            def d_body(d, carry):
                vd = v0 + d
                for c0 in range(8):
                    vec = xva[i32(0), d, pl.ds(c0 * 16, 16)]
                    idx = vd + i32(c0 * 256)
                    plsc.store_scatter(ova, [idx], vec)
                return carry

            lax.fori_loop(i32(0), i32(16), d_body, i32(0))
            wbase = i32(_TAIL_ROW0 * _D) + wid * i32(2048)

            @pl.when(wid < i32(4))
            def _():
                pltpu.sync_copy(ova.at[pl.ds(i32(0), 2048)],
                                fmt_h.at[pl.ds(wbase, 2048)])

            @pl.when(wid == i32(4))
            def _():
                pltpu.sync_copy(ova.at[pl.ds(i32(0), 1024)],
                                fmt_h.at[pl.ds(wbase, 1024)])

    do_table(t2t, f2)
    do_table(t3t, f3)
    do_table(t4t, f4)
    do_tail(tl2, f2)
    do_tail(tl3, f3)
    do_tail(tl4, f4)


@functools.partial(
    pl.kernel,
    out_type=jax.ShapeDtypeStruct((_NROWS, _BLK, _D), jnp.float32),
    mesh=plsc.VectorSubcoreMesh(core_axis_name="c", subcore_axis_name="s"),
    compiler_params=pltpu.CompilerParams(use_tc_tiling_on_sc=False),
    scratch_types=[
        pltpu.VMEM((_NB, _BLK), jnp.int32),
        pltpu.VMEM((_NB, _BLK), jnp.int32),
        pltpu.VMEM((_NB, _BLK), jnp.int32),
        pltpu.VMEM((_BLK, _D), jnp.float32),
        pltpu.VMEM((_BLK, _D), jnp.float32),
        pltpu.VMEM((_BLK, _D), jnp.float32),
        pltpu.VMEM((_BLK, _D), jnp.float32),
        pltpu.VMEM((_BLK, _D), jnp.float32),
        pltpu.VMEM((_BLK, _D), jnp.float32),
        pltpu.SemaphoreType.DMA,
        pltpu.SemaphoreType.DMA,
        pltpu.SemaphoreType.DMA,
        pltpu.SemaphoreType.DMA,
    ],
)
def _sc_embed(h2_h, h3_h, h4_h, t2_h, t3_h, t4_h, out_h,
              vi2, vi3, vi4,
              ra2, ra3, ra4, rb2, rb3, rb4,
              sga, sgb, soa, sob):
    i32 = jnp.int32
    wid = lax.axis_index("s") * i32(_NC) + lax.axis_index("c")
    row0 = wid * i32(_NB)

    pltpu.sync_copy(h2_h.at[pl.ds(row0, _NB)], vi2)
    pltpu.sync_copy(h3_h.at[pl.ds(row0, _NB)], vi3)
    pltpu.sync_copy(h4_h.at[pl.ds(row0, _NB)], vi4)

    def fire_gathers(b, r2, r3, r4, sg):
        pltpu.async_copy(t2_h.at[vi2.at[b]], r2, sg)
        pltpu.async_copy(t3_h.at[vi3.at[b]], r3, sg)
        pltpu.async_copy(t4_h.at[vi4.at[b]], r4, sg)

    def wait_gathers(b, r2, r3, r4, sg):
        pltpu.make_async_copy(t2_h.at[vi2.at[b]], r2, sg).wait()
        pltpu.make_async_copy(t3_h.at[vi3.at[b]], r3, sg).wait()
        pltpu.make_async_copy(t4_h.at[vi4.at[b]], r4, sg).wait()

    def step(b, r2, r3, r4, sg, q2, q3, q4, sq, so_other, so_mine):
        @pl.when(b >= i32(1))
        def _():
            pltpu.make_async_copy(q2, out_h.at[row0 + b - 1], so_other).wait()

        @pl.when(b + 1 < i32(_NB))
        def _():
            fire_gathers(b + 1, q2, q3, q4, sq)

        wait_gathers(b, r2, r3, r4, sg)

        def acc_body(i, carry):
            base = i * i32(8)
            for k in range(8):
                r2[base + k, :] = r2[base + k, :] + r3[base + k, :] + r4[base + k, :]
            return carry

        lax.fori_loop(i32(0), i32(_BLK // 8), acc_body, i32(0))
        pltpu.async_copy(r2, out_h.at[row0 + b], so_mine)

    fire_gathers(i32(0), ra2, ra3, ra4, sga)

    def outer(b2, carry):
        b = b2 * i32(2)
        step(b, ra2, ra3, ra4, sga, rb2, rb3, rb4, sgb, sob, soa)
        step(b + i32(1), rb2, rb3, rb4, sgb, ra2, ra3, ra4, sga, soa, sob)
        return carry

    lax.fori_loop(i32(0), i32(_NB // 2), outer, i32(0))
    pltpu.make_async_copy(rb2, out_h.at[row0 + i32(_NB - 1)], sob).wait()


def kernel(inputs, table_2, table_3, table_4):
    x = inputs.astype(jnp.int32)
    x1 = jnp.pad(x, ((0, 0), (1, 0)))[:, :_S]
    x2 = jnp.pad(x, ((0, 0), (2, 0)))[:, :_S]
    x3 = jnp.pad(x, ((0, 0), (3, 0)))[:, :_S]
    h2, h3, h4 = _hash_tc(x, x1, x2, x3)
    tails = [
        jnp.pad(t[_TAIL_ROW0:].T, ((0, 0), (0, 64)))
        for t in (table_2, table_3, table_4)
    ]
    f2, f3, f4 = _sc_format(
        table_2.T, table_3.T, table_4.T, tails[0], tails[1], tails[2])
    out = _sc_embed(
        h2.reshape(_NROWS, _BLK),
        h3.reshape(_NROWS, _BLK),
        h4.reshape(_NROWS, _BLK),
        f2.reshape(_VOCAB, _D),
        f3.reshape(_VOCAB, _D),
        f4.reshape(_VOCAB, _D),
    )
    return out.reshape(_B, _S, _D)
